# Initial kernel scaffold; baseline (speedup 1.0000x reference)
import jax
import jax.numpy as jnp
from jax.experimental import pallas as pl


def kernel(probs, x_reg, rpn_features_shapes, img_info):
    raise NotImplementedError("write your pallas kernel here")


# speedup vs baseline: 23.8614x; 23.8614x over previous
import numpy as np
import jax
import jax.numpy as jnp
from jax.experimental import pallas as pl
from jax.experimental.pallas import tpu as pltpu

_H_FEAT, _W_FEAT = 50, 76
_STRIDE = 16
_SIZE_BASE = 16
_SCALES = (8.0,)
_RATIOS = (0.5, 1.0, 2.0)
_A = len(_SCALES) * len(_RATIOS)
_N = _H_FEAT * _W_FEAT * _A
_ROWS = 90
_NP = _ROWS * 128
_POST = 2000
_OROWS = 16
_NMS_T = 0.7
_NEG = -1e10

_INTERPRET = False


def _anchor_geometry():
    w0 = float(_SIZE_BASE)
    h0 = float(_SIZE_BASE)
    x_ctr = 0.5 * (w0 - 1.0)
    y_ctr = 0.5 * (h0 - 1.0)
    size = w0 * h0
    base = []
    for r in _RATIOS:
        ws = np.round(np.sqrt(size / r))
        hs = np.round(ws * r)
        for s in _SCALES:
            wss = ws * s
            hss = hs * s
            base.append([x_ctr - 0.5 * (wss - 1.0), y_ctr - 0.5 * (hss - 1.0),
                         x_ctr + 0.5 * (wss - 1.0), y_ctr + 0.5 * (hss - 1.0)])
    ba = np.array(base, np.float32)
    sx = np.arange(_W_FEAT, dtype=np.float32) * _STRIDE
    sy = np.arange(_H_FEAT, dtype=np.float32) * _STRIDE
    mx, my = np.meshgrid(sx, sy)
    shifts = np.stack([mx.ravel(), my.ravel(), mx.ravel(), my.ravel()], axis=1)
    anc = (shifts[:, None, :] + ba[None, :, :]).reshape(-1, 4)
    aw = anc[:, 2] - anc[:, 0] + 1.0
    ah = anc[:, 3] - anc[:, 1] + 1.0
    acx = anc[:, 0] + 0.5 * aw
    acy = anc[:, 1] + 0.5 * ah
    def padp(v, fill):
        out = np.full((_NP,), fill, np.float32)
        out[:_N] = v
        return out.reshape(_ROWS, 128)
    return padp(aw, 1.0), padp(ah, 1.0), padp(acx, 0.0), padp(acy, 0.0)


_AW, _AH, _ACX, _ACY = _anchor_geometry()


def _nms_body(fg_ref, dx_ref, dy_ref, dw_ref, dh_ref,
              aw_ref, ah_ref, acx_ref, acy_ref, info_ref,
              ox1_ref, oy1_ref, ox2_ref, oy2_ref):
    aw = aw_ref[...]
    ah = ah_ref[...]
    cx = acx_ref[...]
    cy = acy_ref[...]
    dx = dx_ref[0]
    dy = dy_ref[0]
    dw = dw_ref[0]
    dh = dh_ref[0]
    bi = pl.program_id(0)
    him = info_ref[bi, 0] - 1.0
    wim = info_ref[bi, 1] - 1.0

    pcx = dx * aw + cx
    pcy = dy * ah + cy
    pw = jnp.exp(dw) * aw
    ph = jnp.exp(dh) * ah
    x1 = jnp.clip(pcx - 0.5 * pw, 0.0, wim)
    y1 = jnp.clip(pcy - 0.5 * ph, 0.0, him)
    x2 = jnp.clip(pcx + 0.5 * pw, 0.0, wim)
    y2 = jnp.clip(pcy + 0.5 * ph, 0.0, him)
    area = (x2 - x1 + 1.0) * (y2 - y1 + 1.0)

    rid = jax.lax.broadcasted_iota(jnp.int32, (_ROWS, 128), 0)
    cid = jax.lax.broadcasted_iota(jnp.int32, (_ROWS, 128), 1)
    lin = rid * 128 + cid
    orid = jax.lax.broadcasted_iota(jnp.int32, (_OROWS, 128), 0)
    ocid = jax.lax.broadcasted_iota(jnp.int32, (_OROWS, 128), 1)
    olin = orid * 128 + ocid

    zout = jnp.zeros((_OROWS, 128), jnp.float32)

    def body(i, carry):
        s, ox1, oy1, ox2, oy2 = carry
        m = jnp.max(s)
        ok = m > _NEG * 0.5
        idx = jnp.min(jnp.where(s == m, lin, _NP))
        sel = lin == idx
        bx1 = jnp.sum(jnp.where(sel, x1, 0.0))
        by1 = jnp.sum(jnp.where(sel, y1, 0.0))
        bx2 = jnp.sum(jnp.where(sel, x2, 0.0))
        by2 = jnp.sum(jnp.where(sel, y2, 0.0))
        barea = jnp.sum(jnp.where(sel, area, 0.0))
        xx1 = jnp.maximum(bx1, x1)
        yy1 = jnp.maximum(by1, y1)
        xx2 = jnp.minimum(bx2, x2)
        yy2 = jnp.minimum(by2, y2)
        inter = (jnp.maximum(xx2 - xx1 + 1.0, 0.0)
                 * jnp.maximum(yy2 - yy1 + 1.0, 0.0))
        iou = inter / (barea + area - inter)
        kill = sel | ((iou > _NMS_T) & ok)
        s = jnp.where(kill, _NEG, s)
        oh = olin == i
        ox1 = jnp.where(oh, jnp.where(ok, bx1, 0.0), ox1)
        oy1 = jnp.where(oh, jnp.where(ok, by1, 0.0), oy1)
        ox2 = jnp.where(oh, jnp.where(ok, bx2, 0.0), ox2)
        oy2 = jnp.where(oh, jnp.where(ok, by2, 0.0), oy2)
        return (s, ox1, oy1, ox2, oy2)

    s0 = fg_ref[0]
    s, ox1, oy1, ox2, oy2 = jax.lax.fori_loop(
        0, _POST, body, (s0, zout, zout, zout, zout))
    ox1_ref[0] = ox1
    oy1_ref[0] = oy1
    ox2_ref[0] = ox2
    oy2_ref[0] = oy2


def kernel(probs, x_reg, rpn_features_shapes, img_info):
    del rpn_features_shapes
    b = probs.shape[0]

    def padplane(v, fill):
        return jnp.pad(v, ((0, 0), (0, _NP - _N)),
                       constant_values=fill).reshape(b, _ROWS, 128)

    fg = padplane(probs[:, :, 0], _NEG)
    dx = padplane(x_reg[:, :, 0], 0.0)
    dy = padplane(x_reg[:, :, 1], 0.0)
    dw = padplane(x_reg[:, :, 2], 0.0)
    dh = padplane(x_reg[:, :, 3], 0.0)

    plane = pl.BlockSpec((1, _ROWS, 128), lambda i: (i, 0, 0))
    aplane = pl.BlockSpec((_ROWS, 128), lambda i: (0, 0))
    oplane = pl.BlockSpec((1, _OROWS, 128), lambda i: (i, 0, 0))
    oshape = jax.ShapeDtypeStruct((b, _OROWS, 128), jnp.float32)

    ox1, oy1, ox2, oy2 = pl.pallas_call(
        _nms_body,
        grid=(b,),
        in_specs=[plane, plane, plane, plane, plane,
                  aplane, aplane, aplane, aplane,
                  pl.BlockSpec((b, 3), lambda i: (0, 0),
                               memory_space=pltpu.SMEM)],
        out_specs=[oplane, oplane, oplane, oplane],
        out_shape=[oshape, oshape, oshape, oshape],
        interpret=_INTERPRET,
    )(fg, dx, dy, dw, dh,
      jnp.asarray(_AW), jnp.asarray(_AH), jnp.asarray(_ACX), jnp.asarray(_ACY),
      img_info)

    sel = jnp.stack([ox1.reshape(b, -1)[:, :_POST],
                     oy1.reshape(b, -1)[:, :_POST],
                     ox2.reshape(b, -1)[:, :_POST],
                     oy2.reshape(b, -1)[:, :_POST]], axis=-1)
    col0 = jnp.broadcast_to(
        jnp.arange(b, dtype=jnp.float32)[:, None, None], (b, _POST, 1))
    return jnp.concatenate([col0, sel], axis=2)


# speedup vs baseline: 58.5987x; 2.4558x over previous
import numpy as np
import jax
import jax.numpy as jnp
from jax.experimental import pallas as pl
from jax.experimental.pallas import tpu as pltpu

_H_FEAT, _W_FEAT = 50, 76
_STRIDE = 16
_SIZE_BASE = 16
_SCALES = (8.0,)
_RATIOS = (0.5, 1.0, 2.0)
_N = _H_FEAT * _W_FEAT * len(_SCALES) * len(_RATIOS)
_ROWS = 90
_NP = _ROWS * 128
_POST = 2000
_OROWS = 16
_NMS_T = 0.7
_NEG = -1e10

_INTERPRET = False


def _anchor_geometry():
    w0 = float(_SIZE_BASE)
    h0 = float(_SIZE_BASE)
    x_ctr = 0.5 * (w0 - 1.0)
    y_ctr = 0.5 * (h0 - 1.0)
    size = w0 * h0
    base = []
    for r in _RATIOS:
        ws = np.round(np.sqrt(size / r))
        hs = np.round(ws * r)
        for s in _SCALES:
            wss = ws * s
            hss = hs * s
            base.append([x_ctr - 0.5 * (wss - 1.0), y_ctr - 0.5 * (hss - 1.0),
                         x_ctr + 0.5 * (wss - 1.0), y_ctr + 0.5 * (hss - 1.0)])
    ba = np.array(base, np.float32)
    sx = np.arange(_W_FEAT, dtype=np.float32) * _STRIDE
    sy = np.arange(_H_FEAT, dtype=np.float32) * _STRIDE
    mx, my = np.meshgrid(sx, sy)
    shifts = np.stack([mx.ravel(), my.ravel(), mx.ravel(), my.ravel()], axis=1)
    anc = (shifts[:, None, :] + ba[None, :, :]).reshape(-1, 4)
    aw = anc[:, 2] - anc[:, 0] + 1.0
    ah = anc[:, 3] - anc[:, 1] + 1.0
    acx = anc[:, 0] + 0.5 * aw
    acy = anc[:, 1] + 0.5 * ah
    def padp(v, fill):
        out = np.full((_NP,), fill, np.float32)
        out[:_N] = v
        return out.reshape(1, _ROWS, 128)
    return padp(aw, 1.0), padp(ah, 1.0), padp(acx, 0.0), padp(acy, 0.0)


_AW, _AH, _ACX, _ACY = _anchor_geometry()


def _rmax(x):
    return jnp.max(jnp.max(x, axis=2, keepdims=True), axis=1, keepdims=True)


def _rmin(x):
    return jnp.min(jnp.min(x, axis=2, keepdims=True), axis=1, keepdims=True)


def _rsum(x):
    return jnp.sum(jnp.sum(x, axis=2, keepdims=True), axis=1, keepdims=True)


def _nms_body(fg_ref, dx_ref, dy_ref, dw_ref, dh_ref,
              aw_ref, ah_ref, acx_ref, acy_ref, wim_ref, him_ref,
              ox1_ref, oy1_ref, ox2_ref, oy2_ref):
    aw = aw_ref[...]
    ah = ah_ref[...]
    cx = acx_ref[...]
    cy = acy_ref[...]
    dx = dx_ref[...]
    dy = dy_ref[...]
    dw = dw_ref[...]
    dh = dh_ref[...]
    wim = wim_ref[...] - 1.0
    him = him_ref[...] - 1.0

    pcx = dx * aw + cx
    pcy = dy * ah + cy
    pw = jnp.exp(dw) * aw
    ph = jnp.exp(dh) * ah
    x1 = jnp.clip(pcx - 0.5 * pw, 0.0, wim)
    y1 = jnp.clip(pcy - 0.5 * ph, 0.0, him)
    x2 = jnp.clip(pcx + 0.5 * pw, 0.0, wim)
    y2 = jnp.clip(pcy + 0.5 * ph, 0.0, him)
    area = (x2 - x1 + 1.0) * (y2 - y1 + 1.0)

    nb = fg_ref.shape[0]
    rid = jax.lax.broadcasted_iota(jnp.int32, (1, _ROWS, 128), 1)
    cid = jax.lax.broadcasted_iota(jnp.int32, (1, _ROWS, 128), 2)
    lin = rid * 128 + cid
    orid = jax.lax.broadcasted_iota(jnp.int32, (1, _OROWS, 128), 1)
    ocid = jax.lax.broadcasted_iota(jnp.int32, (1, _OROWS, 128), 2)
    olin = orid * 128 + ocid

    zout = jnp.zeros((nb, _OROWS, 128), jnp.float32)

    def body(i, carry):
        s, ox1, oy1, ox2, oy2 = carry
        m = _rmax(s)
        ok = m > _NEG * 0.5
        idx = _rmin(jnp.where(s == m, lin, _NP))
        sel = lin == idx
        bx1 = _rsum(jnp.where(sel, x1, 0.0))
        by1 = _rsum(jnp.where(sel, y1, 0.0))
        bx2 = _rsum(jnp.where(sel, x2, 0.0))
        by2 = _rsum(jnp.where(sel, y2, 0.0))
        barea = _rsum(jnp.where(sel, area, 0.0))
        xx1 = jnp.maximum(bx1, x1)
        yy1 = jnp.maximum(by1, y1)
        xx2 = jnp.minimum(bx2, x2)
        yy2 = jnp.minimum(by2, y2)
        inter = (jnp.maximum(xx2 - xx1 + 1.0, 0.0)
                 * jnp.maximum(yy2 - yy1 + 1.0, 0.0))
        iou = inter / (barea + area - inter)
        kill = sel | ((iou > _NMS_T) & ok)
        s = jnp.where(kill, _NEG, s)
        oh = olin == i
        ox1 = jnp.where(oh, jnp.where(ok, bx1, 0.0), ox1)
        oy1 = jnp.where(oh, jnp.where(ok, by1, 0.0), oy1)
        ox2 = jnp.where(oh, jnp.where(ok, bx2, 0.0), ox2)
        oy2 = jnp.where(oh, jnp.where(ok, by2, 0.0), oy2)
        return (s, ox1, oy1, ox2, oy2)

    s0 = fg_ref[...]
    s, ox1, oy1, ox2, oy2 = jax.lax.fori_loop(
        0, _POST, body, (s0, zout, zout, zout, zout))
    ox1_ref[...] = ox1
    oy1_ref[...] = oy1
    ox2_ref[...] = ox2
    oy2_ref[...] = oy2


def kernel(probs, x_reg, rpn_features_shapes, img_info):
    del rpn_features_shapes
    b = probs.shape[0]

    def padplane(v, fill):
        return jnp.pad(v, ((0, 0), (0, _NP - _N)),
                       constant_values=fill).reshape(b, _ROWS, 128)

    fg = padplane(probs[:, :, 0], _NEG)
    dx = padplane(x_reg[:, :, 0], 0.0)
    dy = padplane(x_reg[:, :, 1], 0.0)
    dw = padplane(x_reg[:, :, 2], 0.0)
    dh = padplane(x_reg[:, :, 3], 0.0)
    him = jnp.broadcast_to(img_info[:, 0][:, None, None], (b, 1, 128))
    wim = jnp.broadcast_to(img_info[:, 1][:, None, None], (b, 1, 128))

    oshape = jax.ShapeDtypeStruct((b, _OROWS, 128), jnp.float32)

    ox1, oy1, ox2, oy2 = pl.pallas_call(
        _nms_body,
        out_shape=[oshape, oshape, oshape, oshape],
        interpret=_INTERPRET,
    )(fg, dx, dy, dw, dh,
      jnp.asarray(_AW), jnp.asarray(_AH), jnp.asarray(_ACX), jnp.asarray(_ACY),
      wim, him)

    sel = jnp.stack([ox1.reshape(b, -1)[:, :_POST],
                     oy1.reshape(b, -1)[:, :_POST],
                     ox2.reshape(b, -1)[:, :_POST],
                     oy2.reshape(b, -1)[:, :_POST]], axis=-1)
    col0 = jnp.broadcast_to(
        jnp.arange(b, dtype=jnp.float32)[:, None, None], (b, _POST, 1))
    return jnp.concatenate([col0, sel], axis=2)
